# baseline (speedup 1.0000x reference)
import functools

import jax
import jax.numpy as jnp
from jax import lax
from jax.experimental import pallas as pl
from jax.experimental.pallas import tpu as pltpu
from jax.experimental.pallas import tpu_sc as plsc

_NI = 16384
_NJ = 200
_D = 20
_NC = 2
_NS = 16
_NW = _NC * _NS
_TR = _NJ // 8
_TC = _NI // 128
_TCW = _TC // _NW
_TRPAIRS = _TR // 2
_TW = 1024
_NVEX = 12


def _embed_sc(x4, wf):
    mesh = plsc.VectorSubcoreMesh(core_axis_name="c", subcore_axis_name="s")

    @functools.partial(
        pl.kernel,
        mesh=mesh,
        out_type=jax.ShapeDtypeStruct((_D, _TR, _TC, _TW), jnp.float32),
        scratch_types=[
            pltpu.VMEM((_TR, 1, _TW), jnp.int32),
            pltpu.VMEM((_TR, 1, _TW), jnp.int32),
            pltpu.VMEM((_D, 1, _TW), jnp.float32),
            pltpu.VMEM((_D, 1, _TW), jnp.float32),
            pltpu.VMEM((_NJ,), jnp.float32),
            pltpu.SemaphoreType.DMA,
            pltpu.SemaphoreType.DMA,
            pltpu.SemaphoreType.DMA,
        ],
        compiler_params=pltpu.CompilerParams(
            use_tc_tiling_on_sc=False, needs_layout_passes=False
        ),
    )
    def k(x4_hbm, wf_hbm, out_hbm, xc_a, xc_b, blk_a, blk_b, w_v,
          sem_a, sem_b, sem_x):
        wid = lax.axis_index("s") * _NC + lax.axis_index("c")
        pltpu.sync_copy(wf_hbm, w_v)

        def build(xc, tr, blk):
            lane = lax.iota(jnp.int32, 16)
            colidx = jnp.minimum(lane, 9) * _D
            cols = [
                plsc.load_gather(w_v, [colidx + d]) for d in range(_NVEX)
            ]

            @plsc.parallel_loop(0, _TW // 16, 1, unroll=2)
            def q_body(q):
                start = pl.multiple_of(q * 16, 16)
                xi = xc[tr, 0, pl.ds(start, 16)]
                s = xi * _D
                vs = [
                    cols[d].at[xi].get(mode="promise_in_bounds")
                    for d in range(_NVEX)
                ] + [
                    plsc.load_gather(w_v, [s + d]) for d in range(_NVEX, _D)
                ]
                for d in range(_D):
                    blk[d, 0, pl.ds(start, 16)] = vs[d]

        def fire(tr, tc, blk, sem):
            for d in range(_D):
                pltpu.async_copy(
                    blk.at[d], out_hbm.at[d, tr, pl.ds(tc, 1)], sem
                )

        def drain(blk, sem):
            pltpu.make_async_copy(
                out_hbm.at[:, 0, pl.ds(0, 1)], blk, sem
            ).wait()

        def tc_body(icn, xc):
            tc = wid * _TCW + icn

            def pair_body(p, c2):
                tr = p * 2

                @pl.when(p >= 1)
                def _():
                    drain(blk_a, sem_a)

                build(xc, tr, blk_a)
                fire(tr, tc, blk_a, sem_a)

                @pl.when(p >= 1)
                def _():
                    drain(blk_b, sem_b)

                build(xc, tr + 1, blk_b)
                fire(tr + 1, tc, blk_b, sem_b)
                return c2

            lax.fori_loop(0, _TRPAIRS, pair_body, 0)

            drain(blk_a, sem_a)
            build(xc, _TR - 1, blk_a)
            fire(_TR - 1, tc, blk_a, sem_a)
            drain(blk_a, sem_a)
            drain(blk_b, sem_b)

        tc0 = wid * _TCW
        xcs = [xc_a, xc_b]
        pltpu.sync_copy(x4_hbm.at[:, pl.ds(tc0, 1)], xc_a)
        for icn in range(_TCW):
            cur, nxt = xcs[icn % 2], xcs[(icn + 1) % 2]
            if icn + 1 < _TCW:
                pltpu.async_copy(
                    x4_hbm.at[:, pl.ds(tc0 + icn + 1, 1)], nxt, sem_x
                )
            tc_body(icn, cur)
            if icn + 1 < _TCW:
                pltpu.make_async_copy(
                    x4_hbm.at[:, pl.ds(tc0, 1)], nxt, sem_x
                ).wait()

    return k(x4, wf)


def kernel(x, w):
    x4 = (
        x.astype(jnp.int32)
        .T.reshape(_TR, 8, _TC, 128)
        .transpose(0, 2, 1, 3)
        .reshape(_TR, _TC, _TW)
    )
    wf = w.reshape(-1)
    out5 = _embed_sc(x4, wf)
    out_t = (
        out5.reshape(_D, _TR, _TC, 8, 128)
        .transpose(0, 1, 3, 2, 4)
        .reshape(_D, _NJ, _NI)
    )
    return out_t.transpose(2, 1, 0)


# speedup vs baseline: 1.0367x; 1.0367x over previous
import functools

import jax
import jax.numpy as jnp
from jax import lax
from jax.experimental import pallas as pl
from jax.experimental.pallas import tpu as pltpu
from jax.experimental.pallas import tpu_sc as plsc

_NI = 16384
_NJ = 200
_D = 20
_NC = 2
_NS = 16
_NW = _NC * _NS
_TR = _NJ // 8
_TC = _NI // 128
_TCW = _TC // _NW
_TRPAIRS = _TR // 2
_TW = 1024
_NVEX = 10


def _embed_sc(x4, wf):
    mesh = plsc.VectorSubcoreMesh(core_axis_name="c", subcore_axis_name="s")

    @functools.partial(
        pl.kernel,
        mesh=mesh,
        out_type=jax.ShapeDtypeStruct((_D, _TR, _TC, _TW), jnp.float32),
        scratch_types=[
            pltpu.VMEM((_TR, 1, _TW), jnp.int32),
            pltpu.VMEM((_TR, 1, _TW), jnp.int32),
            pltpu.VMEM((_D, 1, _TW), jnp.float32),
            pltpu.VMEM((_D, 1, _TW), jnp.float32),
            pltpu.VMEM((_NJ,), jnp.float32),
            pltpu.SemaphoreType.DMA,
            pltpu.SemaphoreType.DMA,
            pltpu.SemaphoreType.DMA,
        ],
        compiler_params=pltpu.CompilerParams(
            use_tc_tiling_on_sc=False, needs_layout_passes=False
        ),
    )
    def k(x4_hbm, wf_hbm, out_hbm, xc_a, xc_b, blk_a, blk_b, w_v,
          sem_a, sem_b, sem_x):
        wid = lax.axis_index("s") * _NC + lax.axis_index("c")
        pltpu.sync_copy(wf_hbm, w_v)

        def build(xc, tr, blk):
            lane = lax.iota(jnp.int32, 16)
            colidx = jnp.minimum(lane, 9) * _D
            cols = [
                plsc.load_gather(w_v, [colidx + d]) for d in range(_NVEX)
            ]

            @plsc.parallel_loop(0, _TW // 16, 1, unroll=2)
            def q_body(q):
                start = pl.multiple_of(q * 16, 16)
                xi = xc[tr, 0, pl.ds(start, 16)]
                s = xi * _D
                vs = [
                    cols[d].at[xi].get(mode="promise_in_bounds")
                    for d in range(_NVEX)
                ] + [
                    plsc.load_gather(w_v, [s + d]) for d in range(_NVEX, _D)
                ]
                for d in range(_D):
                    blk[d, 0, pl.ds(start, 16)] = vs[d]

        def fire(tr, tc, blk, sem):
            for d in range(_D):
                pltpu.async_copy(
                    blk.at[d], out_hbm.at[d, tr, pl.ds(tc, 1)], sem
                )

        def drain(blk, sem):
            pltpu.make_async_copy(
                out_hbm.at[:, 0, pl.ds(0, 1)], blk, sem
            ).wait()

        def tc_body(icn, xc):
            tc = wid * _TCW + icn

            def pair_body(p, c2):
                tr = p * 2

                @pl.when(p >= 1)
                def _():
                    drain(blk_a, sem_a)

                build(xc, tr, blk_a)
                fire(tr, tc, blk_a, sem_a)

                @pl.when(p >= 1)
                def _():
                    drain(blk_b, sem_b)

                build(xc, tr + 1, blk_b)
                fire(tr + 1, tc, blk_b, sem_b)
                return c2

            lax.fori_loop(0, _TRPAIRS, pair_body, 0)

            drain(blk_a, sem_a)
            build(xc, _TR - 1, blk_a)
            fire(_TR - 1, tc, blk_a, sem_a)
            drain(blk_a, sem_a)
            drain(blk_b, sem_b)

        tc0 = wid * _TCW
        xcs = [xc_a, xc_b]
        pltpu.sync_copy(x4_hbm.at[:, pl.ds(tc0, 1)], xc_a)
        for icn in range(_TCW):
            cur, nxt = xcs[icn % 2], xcs[(icn + 1) % 2]
            if icn + 1 < _TCW:
                pltpu.async_copy(
                    x4_hbm.at[:, pl.ds(tc0 + icn + 1, 1)], nxt, sem_x
                )
            tc_body(icn, cur)
            if icn + 1 < _TCW:
                pltpu.make_async_copy(
                    x4_hbm.at[:, pl.ds(tc0, 1)], nxt, sem_x
                ).wait()

    return k(x4, wf)


def kernel(x, w):
    x4 = (
        x.astype(jnp.int32)
        .T.reshape(_TR, 8, _TC, 128)
        .transpose(0, 2, 1, 3)
        .reshape(_TR, _TC, _TW)
    )
    wf = w.reshape(-1)
    out5 = _embed_sc(x4, wf)
    out_t = (
        out5.reshape(_D, _TR, _TC, 8, 128)
        .transpose(0, 1, 3, 2, 4)
        .reshape(_D, _NJ, _NI)
    )
    return out_t.transpose(2, 1, 0)
